# Initial kernel scaffold; baseline (speedup 1.0000x reference)
import math

import jax
import jax.numpy as jnp
from jax import lax
from jax.experimental import pallas as pl
from jax.experimental.pallas import tpu as pltpu

_SMOOTH = 0.1
_CONF = 1.0 - _SMOOTH


def _loss_body(t_ref, x_ref, out_ref):
    i = pl.program_id(0)
    bs, v = x_ref.shape
    eps = _SMOOTH / (v - 1)
    k_const = (v - 2) * eps * math.log(eps) + (_CONF + eps) * math.log(_CONF + eps)

    x = x_ref[...]
    t = t_ref[0, pl.ds(i * bs, bs)]
    rowsum = jnp.sum(x, axis=1)
    x0 = x[:, 0]
    col = lax.broadcasted_iota(jnp.int32, x.shape, 1)
    xt = jnp.sum(jnp.where(col == t[:, None], x, 0.0), axis=1)
    contrib = jnp.where(t != 0, k_const - eps * (rowsum - x0) - _CONF * xt, 0.0)

    @pl.when(i == 0)
    def _():
        out_ref[0, 0] = 0.0

    out_ref[0, 0] += jnp.sum(contrib)


def kernel(x, target):
    b, s, v = x.shape
    x2 = x.reshape(b * s, v)
    t2 = target.reshape(1, b * s).astype(jnp.int32)
    bs = 128
    out = pl.pallas_call(
        _loss_body,
        grid=(b * s // bs,),
        in_specs=[
            pl.BlockSpec((1, b * s), lambda i: (0, 0)),
            pl.BlockSpec((bs, v), lambda i: (i, 0)),
        ],
        out_specs=pl.BlockSpec((1, 1), lambda i: (0, 0)),
        out_shape=jax.ShapeDtypeStruct((1, 1), jnp.float32),
        compiler_params=pltpu.CompilerParams(
            dimension_semantics=("arbitrary",),
        ),
    )(t2, x2)
    return out[0, 0]


# speedup vs baseline: 2.9570x; 2.9570x over previous
import math

import jax
import jax.numpy as jnp
from jax import lax
from jax.experimental import pallas as pl
from jax.experimental.pallas import tpu as pltpu

_SMOOTH = 0.1
_CONF = 1.0 - _SMOOTH


def _loss_body(t_ref, x_ref, out_ref):
    i = pl.program_id(0)
    bs, v = x_ref.shape
    eps = _SMOOTH / (v - 1)
    k_const = (v - 2) * eps * math.log(eps) + (_CONF + eps) * math.log(_CONF + eps)

    x = x_ref[...]
    t = t_ref[0, pl.ds(i * bs, bs)]
    rowsum = jnp.sum(x, axis=1)
    x0 = x[:, 0]
    col = lax.broadcasted_iota(jnp.int32, x.shape, 1)
    xt = jnp.sum(jnp.where(col == t[:, None], x, 0.0), axis=1)
    contrib = jnp.where(t != 0, k_const - eps * (rowsum - x0) - _CONF * xt, 0.0)

    @pl.when(i == 0)
    def _():
        out_ref[0, 0] = 0.0

    out_ref[0, 0] += jnp.sum(contrib)


def kernel(x, target):
    b, s, v = x.shape
    x2 = x.reshape(b * s, v)
    t2 = target.reshape(1, b * s).astype(jnp.int32)
    bs = 128
    out = pl.pallas_call(
        _loss_body,
        grid=(b * s // bs,),
        in_specs=[
            pl.BlockSpec((1, b * s), lambda i: (0, 0)),
            pl.BlockSpec((bs, v), lambda i: (i, 0)),
        ],
        out_specs=pl.BlockSpec(memory_space=pltpu.SMEM),
        out_shape=jax.ShapeDtypeStruct((1, 1), jnp.float32),
        compiler_params=pltpu.CompilerParams(
            dimension_semantics=("arbitrary",),
        ),
    )(t2, x2)
    return out[0, 0]
